# baseline (speedup 1.0000x reference)
import functools

import jax
import jax.numpy as jnp
from jax import lax
from jax.experimental import pallas as pl
from jax.experimental.pallas import tpu as pltpu
from jax.experimental.pallas import tpu_sc as plsc

MAXLEN = 200
DIM = 64
LANES = 16
NC, NS = 2, 16
NW = NC * NS
CHUNK = 128


def _pe_table():
    position = jnp.arange(MAXLEN, dtype=jnp.float32)[:, None]
    div_term = jnp.exp(
        jnp.arange(0, DIM, 2, dtype=jnp.float32) * (-jnp.log(10000.0) / DIM)
    )
    pe = jnp.zeros((MAXLEN, (DIM + 1) // 2 * 2), dtype=jnp.float32)
    pe = pe.at[:, 0::2].set(jnp.sin(position * div_term))
    pe = pe.at[:, 1::2].set(jnp.cos(position * div_term))
    return pe[:, :DIM]


def _make_sc_call(n_rows):
    rows_per_w = n_rows // NW
    n_chunks = rows_per_w // CHUNK
    mesh = plsc.VectorSubcoreMesh(core_axis_name="c", subcore_axis_name="s")

    @functools.partial(
        pl.kernel,
        out_type=jax.ShapeDtypeStruct((n_rows, DIM), jnp.float32),
        mesh=mesh,
        scratch_types=[
            pltpu.VMEM((CHUNK,), jnp.int32),
            pltpu.VMEM((CHUNK, DIM), jnp.float32),
            pltpu.VMEM((MAXLEN, DIM), jnp.float32),
            pltpu.SemaphoreType.DMA,
        ],
        compiler_params=pltpu.CompilerParams(use_tc_tiling_on_sc=False),
    )
    def sc_kernel(x_hbm, w_hbm, pe_hbm, out_hbm, idx_v, rows_v, pe_v, sem):
        wid = lax.axis_index("s") * NC + lax.axis_index("c")
        base = wid * rows_per_w
        pltpu.sync_copy(pe_hbm, pe_v)

        @pl.loop(0, n_chunks)
        def _chunk(c):
            off = base + c * CHUNK
            pltpu.sync_copy(x_hbm.at[pl.ds(off, CHUNK)], idx_v)
            pltpu.async_copy(w_hbm.at[idx_v], rows_v, sem).wait()
            pos0 = lax.rem(c * CHUNK, MAXLEN)

            @pl.loop(0, CHUNK)
            def _row(r):
                pos = pos0 + r
                pos = jnp.where(pos >= MAXLEN, pos - MAXLEN, pos)
                for j in range(DIM // LANES):
                    sl = pl.ds(j * LANES, LANES)
                    rows_v[r, sl] = rows_v[r, sl] + pe_v[pos, sl]

            pltpu.sync_copy(rows_v, out_hbm.at[pl.ds(off, CHUNK)])

    return sc_kernel


def kernel(x, W):
    b, t = x.shape
    n_rows = b * t
    pe = _pe_table()
    out = _make_sc_call(n_rows)(x.reshape(n_rows), W, pe)
    return out.reshape(b, t, DIM)


# speedup vs baseline: 1.5569x; 1.5569x over previous
import functools

import jax
import jax.numpy as jnp
from jax import lax
from jax.experimental import pallas as pl
from jax.experimental.pallas import tpu as pltpu
from jax.experimental.pallas import tpu_sc as plsc

MAXLEN = 200
DIM = 64
LANES = 16
NC, NS = 2, 16
NW = NC * NS
NBUF = 4
ROUNDS = MAXLEN // NBUF


def _pe_table():
    position = jnp.arange(MAXLEN, dtype=jnp.float32)[:, None]
    div_term = jnp.exp(
        jnp.arange(0, DIM, 2, dtype=jnp.float32) * (-jnp.log(10000.0) / DIM)
    )
    pe = jnp.zeros((MAXLEN, (DIM + 1) // 2 * 2), dtype=jnp.float32)
    pe = pe.at[:, 0::2].set(jnp.sin(position * div_term))
    pe = pe.at[:, 1::2].set(jnp.cos(position * div_term))
    return pe[:, :DIM]


def _make_sc_call(batch):
    bblk = batch // NW
    mesh = plsc.VectorSubcoreMesh(core_axis_name="c", subcore_axis_name="s")

    @functools.partial(
        pl.kernel,
        out_type=jax.ShapeDtypeStruct((batch, MAXLEN, DIM), jnp.float32),
        mesh=mesh,
        scratch_types=[
            pltpu.VMEM((MAXLEN, bblk), jnp.int32),
            pltpu.VMEM((MAXLEN, DIM), jnp.float32),
            [pltpu.VMEM((bblk, DIM), jnp.float32) for _ in range(NBUF)],
            [pltpu.SemaphoreType.DMA for _ in range(NBUF)],
            [pltpu.SemaphoreType.DMA for _ in range(NBUF)],
        ],
        compiler_params=pltpu.CompilerParams(use_tc_tiling_on_sc=False),
    )
    def sc_kernel(xw_hbm, w_hbm, pe_hbm, out_hbm, idx_v, pe_v, rows, gsem, ssem):
        wid = lax.axis_index("s") * NC + lax.axis_index("c")
        b0 = wid * bblk
        pltpu.sync_copy(xw_hbm.at[wid], idx_v)
        pltpu.sync_copy(pe_hbm, pe_v)

        @pl.loop(0, ROUNDS)
        def _round(g):
            gdesc = []
            for k in range(NBUF):
                t = g * NBUF + k

                @pl.when(g > 0)
                def _():
                    pltpu.make_async_copy(
                        rows[k], out_hbm.at[pl.ds(b0, bblk), 0], ssem[k]
                    ).wait()

                gdesc.append(
                    pltpu.async_copy(w_hbm.at[idx_v.at[t]], rows[k], gsem[k])
                )
            for k in range(NBUF):
                t = g * NBUF + k
                gdesc[k].wait()
                pvec = [pe_v[t, pl.ds(j * LANES, LANES)] for j in range(DIM // LANES)]

                @pl.loop(0, bblk, unroll=8)
                def _row(r):
                    for j in range(DIM // LANES):
                        plsc.addupdate(rows[k].at[r, pl.ds(j * LANES, LANES)], pvec[j])

                pltpu.async_copy(
                    rows[k], out_hbm.at[pl.ds(b0, bblk), t], ssem[k]
                )

        for k in range(NBUF):
            pltpu.make_async_copy(
                rows[k], out_hbm.at[pl.ds(b0, bblk), 0], ssem[k]
            ).wait()

    return sc_kernel


def kernel(x, W):
    b, t = x.shape
    pe = _pe_table()
    xw = x.reshape(NW, b // NW, t).transpose(0, 2, 1)
    return _make_sc_call(b)(xw, W, pe)
